# baseline (speedup 1.0000x reference)
import jax
import jax.numpy as jnp
from jax.experimental import pallas as pl

R_BLK = 128


def _body(x_ref, pos_ref, out_ref):
    out_ref[...] = x_ref[...] + pos_ref[...]


def kernel(x, pos_table):
    B, n, d = x.shape
    R = n * d
    x2 = jnp.transpose(x, (1, 2, 0)).reshape(R, B)
    pos2 = pos_table[:n].reshape(R, 1)
    out2 = pl.pallas_call(
        _body,
        grid=(R // R_BLK,),
        in_specs=[
            pl.BlockSpec((R_BLK, B), lambda i: (i, 0)),
            pl.BlockSpec((R_BLK, 1), lambda i: (i, 0)),
        ],
        out_specs=pl.BlockSpec((R_BLK, B), lambda i: (i, 0)),
        out_shape=jax.ShapeDtypeStruct((R, B), x.dtype),
    )(x2, pos2)
    return jnp.transpose(out2.reshape(n, d, B), (2, 0, 1))


# speedup vs baseline: 1.1041x; 1.1041x over previous
import jax
import jax.numpy as jnp
from jax.experimental import pallas as pl

R_BLK = 640


def _body(x_ref, pos_ref, out_ref):
    out_ref[...] = x_ref[...] + pos_ref[...]


def kernel(x, pos_table):
    B, n, d = x.shape
    R = n * d
    x2 = jnp.transpose(x, (1, 2, 0)).reshape(R, B)
    pos2 = pos_table[:n].reshape(R, 1)
    out2 = pl.pallas_call(
        _body,
        grid=(R // R_BLK,),
        in_specs=[
            pl.BlockSpec((R_BLK, B), lambda i: (i, 0)),
            pl.BlockSpec((R_BLK, 1), lambda i: (i, 0)),
        ],
        out_specs=pl.BlockSpec((R_BLK, B), lambda i: (i, 0)),
        out_shape=jax.ShapeDtypeStruct((R, B), x.dtype),
    )(x2, pos2)
    return jnp.transpose(out2.reshape(n, d, B), (2, 0, 1))
